# Initial kernel scaffold; baseline (speedup 1.0000x reference)
import jax
import jax.numpy as jnp
from jax.experimental import pallas as pl


def kernel(x, edge_index, batch, W1, b1, W2, b2, Wl, bl):
    raise NotImplementedError("write your pallas kernel here")


# speedup vs baseline: 14.5467x; 14.5467x over previous
import functools

import jax
import jax.numpy as jnp
from jax import lax
from jax.experimental import pallas as pl
from jax.experimental.pallas import tpu as pltpu
from jax.experimental.pallas import tpu_sc as plsc

N = 10000
E = 320000
D = 128
H = 128
G = 64

NC = 2
NS = 16
NW = NC * NS
EB = 128
NB = E // EB
NPAD = 10240
RPT = NPAD // NS
DEGW = 128

_mesh = plsc.VectorSubcoreMesh(core_axis_name="c", subcore_axis_name="s")


def _wid():
    return lax.axis_index("s") * NC + lax.axis_index("c")


def _nbatches(wid):
    return jnp.where(wid < NB - (NB // NW) * NW, NB // NW + 1, NB // NW)


@functools.partial(
    pl.kernel,
    mesh=_mesh,
    out_type=jax.ShapeDtypeStruct((NC, NPAD, DEGW), jnp.float32),
    scratch_types=[
        pltpu.VMEM((EB, DEGW), jnp.float32),
        pltpu.VMEM((EB,), jnp.int32),
        pltpu.VMEM_SHARED((NPAD, DEGW), jnp.float32),
    ],
)
def _deg_kernel(dst_hbm, ones_hbm, zeros_hbm, cnt_hbm, ones_v, idx_v, acc):
    c = lax.axis_index("c")
    s = lax.axis_index("s")
    wid = _wid()
    base = s * RPT
    for k in range(RPT // EB):
        pltpu.sync_copy(zeros_hbm, acc.at[pl.ds(base + k * EB, EB)])
    pltpu.sync_copy(ones_hbm, ones_v)
    plsc.subcore_barrier()

    def body(j, carry):
        g = wid + j * NW
        pltpu.sync_copy(dst_hbm.at[g], idx_v)
        pltpu.sync_copy(ones_v, acc.at[idx_v], add=True)
        return carry

    lax.fori_loop(0, _nbatches(wid), body, 0)
    plsc.subcore_barrier()
    for k in range(RPT // EB):
        sl = pl.ds(base + k * EB, EB)
        pltpu.sync_copy(acc.at[sl], cnt_hbm.at[c, sl])


@functools.partial(
    pl.kernel,
    mesh=_mesh,
    out_type=jax.ShapeDtypeStruct((NC, NPAD, H), jnp.float32),
    scratch_types=[
        pltpu.VMEM((EB,), jnp.int32),
        pltpu.VMEM((EB,), jnp.int32),
        pltpu.VMEM((EB, H), jnp.float32),
        pltpu.VMEM_SHARED((NPAD, H), jnp.float32),
        pltpu.SemaphoreType.DMA,
    ],
)
def _scat_kernel(z_hbm, src_hbm, dst_hbm, zeros_hbm, out_hbm,
                 sidx, didx, rows, acc, sem):
    c = lax.axis_index("c")
    s = lax.axis_index("s")
    wid = _wid()
    base = s * RPT
    for k in range(RPT // EB):
        pltpu.sync_copy(zeros_hbm, acc.at[pl.ds(base + k * EB, EB)])
    plsc.subcore_barrier()

    def body(j, carry):
        g = wid + j * NW
        pltpu.sync_copy(src_hbm.at[g], sidx)
        pltpu.sync_copy(dst_hbm.at[g], didx)
        pltpu.async_copy(z_hbm.at[sidx], rows, sem).wait()
        pltpu.sync_copy(rows, acc.at[didx], add=True)
        return carry

    lax.fori_loop(0, _nbatches(wid), body, 0)
    plsc.subcore_barrier()
    for k in range(RPT // EB):
        sl = pl.ds(base + k * EB, EB)
        pltpu.sync_copy(acc.at[sl], out_hbm.at[c, sl])


def _mm1_body(x_ref, w_ref, cnt_ref, z_ref, dinv_ref):
    cnt = cnt_ref[0, :N, 0:1] + cnt_ref[1, :N, 0:1]
    dinv = lax.rsqrt(cnt + 1.0)
    dinv_ref[...] = dinv
    xw = jnp.dot(x_ref[...], w_ref[...], preferred_element_type=jnp.float32)
    z_ref[...] = xw * dinv


_mm1 = functools.partial(
    pl.pallas_call,
    out_shape=(jax.ShapeDtypeStruct((N, H), jnp.float32),
               jax.ShapeDtypeStruct((N, 1), jnp.float32)),
)(_mm1_body)


def _mid_body(p_ref, z1_ref, dinv_ref, w2_ref, b1_ref, z2_ref):
    s = p_ref[0, :N, :] + p_ref[1, :N, :]
    dinv = dinv_ref[...]
    h = jnp.maximum((s + z1_ref[...]) * dinv + b1_ref[...], 0.0)
    z2_ref[...] = jnp.dot(
        h, w2_ref[...], preferred_element_type=jnp.float32) * dinv


_mid = functools.partial(
    pl.pallas_call,
    out_shape=jax.ShapeDtypeStruct((N, H), jnp.float32),
)(_mid_body)


def _fin_body(p_ref, z2_ref, dinv_ref, b2_ref, batch_ref, wl_ref, bl_ref,
              out_ref):
    s = p_ref[0, :N, :] + p_ref[1, :N, :]
    h = (s + z2_ref[...]) * dinv_ref[...] + b2_ref[...]
    b = batch_ref[...]
    gids = lax.broadcasted_iota(jnp.int32, (G, N), 0)
    onehot = (b == gids).astype(jnp.float32)
    sums = jnp.dot(onehot, h, preferred_element_type=jnp.float32)
    cnts = jnp.sum(onehot, axis=1, keepdims=True)
    pooled = sums / jnp.maximum(cnts, 1.0)
    out_ref[...] = jnp.dot(
        pooled, wl_ref[...], preferred_element_type=jnp.float32) + bl_ref[...]


_fin = functools.partial(
    pl.pallas_call,
    out_shape=jax.ShapeDtypeStruct((G, 1), jnp.float32),
)(_fin_body)


def kernel(x, edge_index, batch, W1, b1, W2, b2, Wl, bl):
    src = edge_index[0].reshape(NB, EB)
    dst = edge_index[1].reshape(NB, EB)
    ones16 = jnp.ones((EB, DEGW), jnp.float32)
    zeros16 = jnp.zeros((EB, DEGW), jnp.float32)
    zeros128 = jnp.zeros((EB, H), jnp.float32)

    cnt = _deg_kernel(dst, ones16, zeros16)
    z1, dinv = _mm1(x, W1, cnt)
    p1 = _scat_kernel(z1, src, dst, zeros128)
    z2 = _mid(p1, z1, dinv, W2, b1.reshape(1, H))
    p2 = _scat_kernel(z2, src, dst, zeros128)
    return _fin(p2, z2, dinv, b2.reshape(1, H), batch.reshape(1, N),
                Wl, bl.reshape(1, 1))
